# baseline (speedup 1.0000x reference)
import functools

import jax
import jax.numpy as jnp
import numpy as np
from jax import lax
from jax.experimental import pallas as pl
from jax.experimental.pallas import tpu as pltpu
from jax.experimental.pallas import tpu_sc as plsc

_BINS = 10
_EDGES = np.log(np.arange(1, _BINS, dtype=np.float64) / _BINS
                / (1.0 - np.arange(1, _BINS, dtype=np.float64) / _BINS)
                ).astype(np.float32)

_LOG1P = np.array([
    5.629329962175689e-07, 0.9999574422836304, -0.49920639395713806,
    0.3269723653793335, -0.2228347212076187, 0.13076335191726685,
    -0.05262395367026329, 0.01011890172958374], dtype=np.float32)

_ROWS = 4000
_CHUNK = 8

_SC_TILES = 32
_SC_TILE_ROWS = 7000
_SC_ROWS = _SC_TILES * _SC_TILE_ROWS
_SC_CHUNK = 200


def _tc_kernel(pred_ref, tgt_ref, out_ref, acc_ref, *, nsteps):
    step = pl.program_id(0)

    @pl.when(step == 0)
    def _init():
        acc_ref[...] = jnp.zeros_like(acc_ref)

    n_cols = pred_ref.shape[1]
    zero = jnp.zeros((_CHUNK, n_cols), jnp.float32)
    acc_c = [zero] * 9
    acc_s = [zero] * 9
    acc_t = zero
    for k in range(_ROWS // _CHUNK):
        p = pred_ref[pl.ds(k * _CHUNK, _CHUNK), :]
        ti = tgt_ref[pl.ds(k * _CHUNK, _CHUNK), :]
        q = jax.lax.bitcast_convert_type(
            jax.lax.bitcast_convert_type(p, jnp.int32) ^ (ti << 31),
            jnp.float32)
        bce = jnp.maximum(q, 0.0) + jnp.log1p(jnp.exp(-jnp.abs(p)))
        for i in range(9):
            mf = (q >= _EDGES[i]).astype(jnp.float32)
            acc_c[i] = acc_c[i] + mf
            acc_s[i] = acc_s[i] + mf * bce
        acc_t = acc_t + bce
    rows = ([acc_ref[i] + jnp.sum(acc_c[i], axis=0) for i in range(9)]
            + [acc_ref[9 + i] + jnp.sum(acc_s[i], axis=0) for i in range(9)]
            + [acc_ref[18] + jnp.sum(acc_t, axis=0)])
    acc_ref[...] = jnp.stack(rows, axis=0)

    @pl.when(step == nsteps - 1)
    def _finish():
        out_ref[...] = acc_ref[...]


def _sc_kernel(pred_hbm, tgt_hbm, out_hbm, pchunk, tchunk, accv, *, row0):
    cid = lax.axis_index("c")
    sid = lax.axis_index("s")
    wid = sid * 2 + cid
    tile_row0 = row0 + wid * _SC_TILE_ROWS
    accs = tuple([jnp.zeros((16,), jnp.float32)] * 19)
    for ch in range(_SC_TILE_ROWS // _SC_CHUNK):
        base = tile_row0 + ch * _SC_CHUNK
        pltpu.sync_copy(pred_hbm.at[pl.ds(base, _SC_CHUNK)], pchunk)
        pltpu.sync_copy(tgt_hbm.at[pl.ds(base, _SC_CHUNK)], tchunk)

        def body(g, accs_t):
            acc = list(accs_t)
            r = g // 5
            c = (g % 5) * 16
            p = pchunk[r, pl.ds(c, 16)]
            ti = tchunk[r, pl.ds(c, 16)]
            q = jax.lax.bitcast_convert_type(
                jax.lax.bitcast_convert_type(p, jnp.int32) ^ (ti << 31),
                jnp.float32)
            e = jnp.exp(-jnp.abs(p))
            l = e * float(_LOG1P[7]) + float(_LOG1P[6])
            for d in range(5, -1, -1):
                l = l * e + float(_LOG1P[d])
            bce = jnp.maximum(q, 0.0) + l
            for i in range(9):
                m = q >= _EDGES[i]
                acc[i] = acc[i] + jnp.where(m, 1.0, 0.0)
                acc[9 + i] = acc[9 + i] + jnp.where(m, bce, 0.0)
            acc[18] = acc[18] + bce
            return tuple(acc)

        accs = lax.fori_loop(0, _SC_CHUNK * 5, body, accs)
    for i in range(19):
        accv[i] = accs[i]
    pltpu.sync_copy(accv, out_hbm.at[wid])


def kernel(pred, target):
    n_rows, n_cols = pred.shape
    tc_rows = n_rows - _SC_ROWS
    assert tc_rows % _ROWS == 0
    nsteps = tc_rows // _ROWS
    tc_part = pl.pallas_call(
        functools.partial(_tc_kernel, nsteps=nsteps),
        grid=(nsteps,),
        in_specs=[
            pl.BlockSpec((_ROWS, n_cols), lambda i: (i, 0)),
            pl.BlockSpec((_ROWS, n_cols), lambda i: (i, 0)),
        ],
        out_specs=pl.BlockSpec((19, n_cols), lambda i: (0, 0)),
        out_shape=jax.ShapeDtypeStruct((19, n_cols), jnp.float32),
        scratch_shapes=[pltpu.VMEM((19, n_cols), jnp.float32)],
    )(pred, target)

    sc_fn = pl.kernel(
        functools.partial(_sc_kernel, row0=tc_rows),
        out_type=jax.ShapeDtypeStruct((_SC_TILES, 19, 16), jnp.float32),
        mesh=plsc.VectorSubcoreMesh(core_axis_name="c", subcore_axis_name="s"),
        scratch_types=[
            pltpu.VMEM((_SC_CHUNK, n_cols), jnp.float32),
            pltpu.VMEM((_SC_CHUNK, n_cols), jnp.int32),
            pltpu.VMEM((19, 16), jnp.float32),
        ],
    )
    sc_part = sc_fn(pred, target)

    acc = jnp.sum(tc_part, axis=1) + jnp.sum(sc_part, axis=(0, 2))
    total = jnp.float32(n_rows * n_cols)
    c = jnp.concatenate([total[None], acc[0:9], jnp.zeros((1,), jnp.float32)])
    s = jnp.concatenate([acc[18:19], acc[9:18], jnp.zeros((1,), jnp.float32)])
    counts = c[:-1] - c[1:]
    sums = s[:-1] - s[1:]
    n = jnp.sum((counts > 0.0).astype(jnp.float32))
    loss = jnp.sum(jnp.where(counts > 0.0,
                             sums / (jnp.maximum(counts, 1.0) * n), 0.0))
    return loss


# speedup vs baseline: 1.2044x; 1.2044x over previous
import functools

import jax
import jax.numpy as jnp
import numpy as np
from jax import lax
from jax.experimental import pallas as pl
from jax.experimental.pallas import tpu as pltpu
from jax.experimental.pallas import tpu_sc as plsc

_BINS = 10
_EDGES = np.log(np.arange(1, _BINS, dtype=np.float64) / _BINS
                / (1.0 - np.arange(1, _BINS, dtype=np.float64) / _BINS)
                ).astype(np.float32)

_LOG1P = np.array([
    5.629329962175689e-07, 0.9999574422836304, -0.49920639395713806,
    0.3269723653793335, -0.2228347212076187, 0.13076335191726685,
    -0.05262395367026329, 0.01011890172958374], dtype=np.float32)

_ROWS = 4000
_CHUNK = 8

_SC_TILES = 32
_SC_TILE_ROWS = 5000
_SC_ROWS = _SC_TILES * _SC_TILE_ROWS
_SC_CHUNK = 200


def _tc_kernel(pred_ref, tgt_ref, out_ref, acc_ref, *, nsteps):
    step = pl.program_id(0)

    @pl.when(step == 0)
    def _init():
        acc_ref[...] = jnp.zeros_like(acc_ref)

    n_cols = pred_ref.shape[1]
    zero = jnp.zeros((_CHUNK, n_cols), jnp.float32)
    acc_c = [zero] * 9
    acc_s = [zero] * 9
    acc_t = zero
    for k in range(_ROWS // _CHUNK):
        p = pred_ref[pl.ds(k * _CHUNK, _CHUNK), :]
        ti = tgt_ref[pl.ds(k * _CHUNK, _CHUNK), :]
        q = jax.lax.bitcast_convert_type(
            jax.lax.bitcast_convert_type(p, jnp.int32) ^ (ti << 31),
            jnp.float32)
        bce = jnp.maximum(q, 0.0) + jnp.log1p(jnp.exp(-jnp.abs(p)))
        for i in range(9):
            mf = (q >= _EDGES[i]).astype(jnp.float32)
            acc_c[i] = acc_c[i] + mf
            acc_s[i] = acc_s[i] + mf * bce
        acc_t = acc_t + bce
    rows = ([acc_ref[i] + jnp.sum(acc_c[i], axis=0) for i in range(9)]
            + [acc_ref[9 + i] + jnp.sum(acc_s[i], axis=0) for i in range(9)]
            + [acc_ref[18] + jnp.sum(acc_t, axis=0)])
    acc_ref[...] = jnp.stack(rows, axis=0)

    @pl.when(step == nsteps - 1)
    def _finish():
        out_ref[...] = acc_ref[...]


def _sc_kernel(pred_hbm, tgt_hbm, out_hbm, pchunk, tchunk, accv, *, row0):
    cid = lax.axis_index("c")
    sid = lax.axis_index("s")
    wid = sid * 2 + cid
    tile_row0 = row0 + wid * _SC_TILE_ROWS
    accs = tuple([jnp.zeros((16,), jnp.float32)] * 19)
    for ch in range(_SC_TILE_ROWS // _SC_CHUNK):
        base = tile_row0 + ch * _SC_CHUNK
        pltpu.sync_copy(pred_hbm.at[pl.ds(base, _SC_CHUNK)], pchunk)
        pltpu.sync_copy(tgt_hbm.at[pl.ds(base, _SC_CHUNK)], tchunk)

        def body(g, accs_t):
            acc = list(accs_t)
            r = g // 5
            c = (g % 5) * 16
            p = pchunk[r, pl.ds(c, 16)]
            ti = tchunk[r, pl.ds(c, 16)]
            q = jax.lax.bitcast_convert_type(
                jax.lax.bitcast_convert_type(p, jnp.int32) ^ (ti << 31),
                jnp.float32)
            e = jnp.exp(-jnp.abs(p))
            l = e * float(_LOG1P[7]) + float(_LOG1P[6])
            for d in range(5, -1, -1):
                l = l * e + float(_LOG1P[d])
            bce = jnp.maximum(q, 0.0) + l
            for i in range(9):
                m = q >= _EDGES[i]
                acc[i] = acc[i] + jnp.where(m, 1.0, 0.0)
                acc[9 + i] = acc[9 + i] + jnp.where(m, bce, 0.0)
            acc[18] = acc[18] + bce
            return tuple(acc)

        accs = lax.fori_loop(0, _SC_CHUNK * 5, body, accs)
    for i in range(19):
        accv[i] = accs[i]
    pltpu.sync_copy(accv, out_hbm.at[wid])


def kernel(pred, target):
    n_rows, n_cols = pred.shape
    tc_rows = n_rows - _SC_ROWS
    assert tc_rows % _ROWS == 0
    nsteps = tc_rows // _ROWS
    tc_part = pl.pallas_call(
        functools.partial(_tc_kernel, nsteps=nsteps),
        grid=(nsteps,),
        in_specs=[
            pl.BlockSpec((_ROWS, n_cols), lambda i: (i, 0)),
            pl.BlockSpec((_ROWS, n_cols), lambda i: (i, 0)),
        ],
        out_specs=pl.BlockSpec((19, n_cols), lambda i: (0, 0)),
        out_shape=jax.ShapeDtypeStruct((19, n_cols), jnp.float32),
        scratch_shapes=[pltpu.VMEM((19, n_cols), jnp.float32)],
    )(pred, target)

    sc_fn = pl.kernel(
        functools.partial(_sc_kernel, row0=tc_rows),
        out_type=jax.ShapeDtypeStruct((_SC_TILES, 19, 16), jnp.float32),
        mesh=plsc.VectorSubcoreMesh(core_axis_name="c", subcore_axis_name="s"),
        scratch_types=[
            pltpu.VMEM((_SC_CHUNK, n_cols), jnp.float32),
            pltpu.VMEM((_SC_CHUNK, n_cols), jnp.int32),
            pltpu.VMEM((19, 16), jnp.float32),
        ],
    )
    sc_part = sc_fn(pred, target)

    acc = jnp.sum(tc_part, axis=1) + jnp.sum(sc_part, axis=(0, 2))
    total = jnp.float32(n_rows * n_cols)
    c = jnp.concatenate([total[None], acc[0:9], jnp.zeros((1,), jnp.float32)])
    s = jnp.concatenate([acc[18:19], acc[9:18], jnp.zeros((1,), jnp.float32)])
    counts = c[:-1] - c[1:]
    sums = s[:-1] - s[1:]
    n = jnp.sum((counts > 0.0).astype(jnp.float32))
    loss = jnp.sum(jnp.where(counts > 0.0,
                             sums / (jnp.maximum(counts, 1.0) * n), 0.0))
    return loss


# speedup vs baseline: 1.2559x; 1.0427x over previous
import functools

import jax
import jax.numpy as jnp
import numpy as np
from jax import lax
from jax.experimental import pallas as pl
from jax.experimental.pallas import tpu as pltpu
from jax.experimental.pallas import tpu_sc as plsc

_BINS = 10
_EDGES = np.log(np.arange(1, _BINS, dtype=np.float64) / _BINS
                / (1.0 - np.arange(1, _BINS, dtype=np.float64) / _BINS)
                ).astype(np.float32)

_LOG1P = np.array([
    5.629329962175689e-07, 0.9999574422836304, -0.49920639395713806,
    0.3269723653793335, -0.2228347212076187, 0.13076335191726685,
    -0.05262395367026329, 0.01011890172958374], dtype=np.float32)

_ROWS = 4200
_CHUNK = 8

_SC_TILES = 32
_SC_TILE_ROWS = 4600
_SC_ROWS = _SC_TILES * _SC_TILE_ROWS
_SC_CHUNK = 200


def _tc_kernel(pred_ref, tgt_ref, out_ref, acc_ref, *, nsteps):
    step = pl.program_id(0)

    @pl.when(step == 0)
    def _init():
        acc_ref[...] = jnp.zeros_like(acc_ref)

    n_cols = pred_ref.shape[1]
    zero = jnp.zeros((_CHUNK, n_cols), jnp.float32)
    acc_c = [zero] * 9
    acc_s = [zero] * 9
    acc_t = zero
    for k in range(_ROWS // _CHUNK):
        p = pred_ref[pl.ds(k * _CHUNK, _CHUNK), :]
        ti = tgt_ref[pl.ds(k * _CHUNK, _CHUNK), :]
        q = jax.lax.bitcast_convert_type(
            jax.lax.bitcast_convert_type(p, jnp.int32) ^ (ti << 31),
            jnp.float32)
        bce = jnp.maximum(q, 0.0) + jnp.log1p(jnp.exp(-jnp.abs(p)))
        for i in range(9):
            mf = (q >= _EDGES[i]).astype(jnp.float32)
            acc_c[i] = acc_c[i] + mf
            acc_s[i] = acc_s[i] + mf * bce
        acc_t = acc_t + bce
    rows = ([acc_ref[i] + jnp.sum(acc_c[i], axis=0) for i in range(9)]
            + [acc_ref[9 + i] + jnp.sum(acc_s[i], axis=0) for i in range(9)]
            + [acc_ref[18] + jnp.sum(acc_t, axis=0)])
    acc_ref[...] = jnp.stack(rows, axis=0)

    @pl.when(step == nsteps - 1)
    def _finish():
        out_ref[...] = acc_ref[...]


def _sc_kernel(pred_hbm, tgt_hbm, out_hbm, pchunk, tchunk, accv, *, row0):
    cid = lax.axis_index("c")
    sid = lax.axis_index("s")
    wid = sid * 2 + cid
    tile_row0 = row0 + wid * _SC_TILE_ROWS
    accs = tuple([jnp.zeros((16,), jnp.float32)] * 19)
    for ch in range(_SC_TILE_ROWS // _SC_CHUNK):
        base = tile_row0 + ch * _SC_CHUNK
        pltpu.sync_copy(pred_hbm.at[pl.ds(base, _SC_CHUNK)], pchunk)
        pltpu.sync_copy(tgt_hbm.at[pl.ds(base, _SC_CHUNK)], tchunk)

        def body(g, accs_t):
            acc = list(accs_t)
            r = g // 5
            c = (g % 5) * 16
            p = pchunk[r, pl.ds(c, 16)]
            ti = tchunk[r, pl.ds(c, 16)]
            q = jax.lax.bitcast_convert_type(
                jax.lax.bitcast_convert_type(p, jnp.int32) ^ (ti << 31),
                jnp.float32)
            e = jnp.exp(-jnp.abs(p))
            l = e * float(_LOG1P[7]) + float(_LOG1P[6])
            for d in range(5, -1, -1):
                l = l * e + float(_LOG1P[d])
            bce = jnp.maximum(q, 0.0) + l
            for i in range(9):
                m = q >= _EDGES[i]
                acc[i] = acc[i] + jnp.where(m, 1.0, 0.0)
                acc[9 + i] = acc[9 + i] + jnp.where(m, bce, 0.0)
            acc[18] = acc[18] + bce
            return tuple(acc)

        accs = lax.fori_loop(0, _SC_CHUNK * 5, body, accs)
    for i in range(19):
        accv[i] = accs[i]
    pltpu.sync_copy(accv, out_hbm.at[wid])


def kernel(pred, target):
    n_rows, n_cols = pred.shape
    tc_rows = n_rows - _SC_ROWS
    assert tc_rows % _ROWS == 0
    nsteps = tc_rows // _ROWS
    tc_part = pl.pallas_call(
        functools.partial(_tc_kernel, nsteps=nsteps),
        grid=(nsteps,),
        in_specs=[
            pl.BlockSpec((_ROWS, n_cols), lambda i: (i, 0)),
            pl.BlockSpec((_ROWS, n_cols), lambda i: (i, 0)),
        ],
        out_specs=pl.BlockSpec((19, n_cols), lambda i: (0, 0)),
        out_shape=jax.ShapeDtypeStruct((19, n_cols), jnp.float32),
        scratch_shapes=[pltpu.VMEM((19, n_cols), jnp.float32)],
    )(pred, target)

    sc_fn = pl.kernel(
        functools.partial(_sc_kernel, row0=tc_rows),
        out_type=jax.ShapeDtypeStruct((_SC_TILES, 19, 16), jnp.float32),
        mesh=plsc.VectorSubcoreMesh(core_axis_name="c", subcore_axis_name="s"),
        scratch_types=[
            pltpu.VMEM((_SC_CHUNK, n_cols), jnp.float32),
            pltpu.VMEM((_SC_CHUNK, n_cols), jnp.int32),
            pltpu.VMEM((19, 16), jnp.float32),
        ],
    )
    sc_part = sc_fn(pred, target)

    acc = jnp.sum(tc_part, axis=1) + jnp.sum(sc_part, axis=(0, 2))
    total = jnp.float32(n_rows * n_cols)
    c = jnp.concatenate([total[None], acc[0:9], jnp.zeros((1,), jnp.float32)])
    s = jnp.concatenate([acc[18:19], acc[9:18], jnp.zeros((1,), jnp.float32)])
    counts = c[:-1] - c[1:]
    sums = s[:-1] - s[1:]
    n = jnp.sum((counts > 0.0).astype(jnp.float32))
    loss = jnp.sum(jnp.where(counts > 0.0,
                             sums / (jnp.maximum(counts, 1.0) * n), 0.0))
    return loss
